# Initial kernel scaffold; baseline (speedup 1.0000x reference)
import jax
import jax.numpy as jnp
from jax.experimental import pallas as pl


def kernel(job_done, machine_busy_until, truck_location, job_ops_adj, op_scheduled, next_op, proc_times, ops_ma_adj, truck_busy_until, action_mask):
    raise NotImplementedError("write your pallas kernel here")


# speedup vs baseline: 1.5388x; 1.5388x over previous
import functools

import jax
import jax.numpy as jnp
from jax import lax
from jax.experimental import pallas as pl
from jax.experimental.pallas import tpu as pltpu
from jax.experimental.pallas import tpu_sc as plsc

_B = 2048
_NJ = 50
_NM = 20
_NT = 10
_NO = 500
_NACT = 1 + _NJ * _NM * _NT

_NC = 2
_NS = 16
_L = 16
_NW = _NC * _NS
_BW = _B // _NW
_NG = _BW // _L

_BB = 64
_BL = 64


def _job_body(m_ref, a_ref, nop_ref, op_ref, j_ref):
    a = a_ref[...]
    ahi = a.astype(jnp.bfloat16)
    alo = (a - ahi.astype(jnp.float32)).astype(jnp.bfloat16)
    mb = m_ref[...].astype(jnp.bfloat16)
    dn = (((1,), (1,)), ((), ()))
    c = (lax.dot_general(mb, ahi, dn, preferred_element_type=jnp.float32)
         + lax.dot_general(mb, alo, dn, preferred_element_type=jnp.float32))
    rem = jnp.concatenate(
        [lax.slice(c, (b, _NJ * b), (b + 1, _NJ * b + _NJ)) for b in range(_BB)],
        axis=0,
    )
    maxv = jnp.max(rem, axis=1, keepdims=True)
    jid = lax.broadcasted_iota(jnp.int32, (_BB, _NJ), 1)
    selj = jnp.min(jnp.where(rem == maxv, jid, _NJ), axis=1, keepdims=True)
    opv = jnp.sum(jnp.where(jid == selj, nop_ref[...], 0), axis=1, keepdims=True)
    op_ref[...] = jnp.broadcast_to(opv, (_BB, 128))
    j_ref[...] = jnp.broadcast_to(selj, (_BB, 128))


def _job_select(mask_f, adj2, next_op):
    out = pl.pallas_call(
        _job_body,
        grid=(_B // _BB,),
        in_specs=[
            pl.BlockSpec((_BB, _NO), lambda i: (i, 0)),
            pl.BlockSpec((_BB * _NJ, _NO), lambda i: (i, 0)),
            pl.BlockSpec((_BB, _NJ), lambda i: (i, 0)),
        ],
        out_specs=[
            pl.BlockSpec((_BB, 128), lambda i: (i, 0)),
            pl.BlockSpec((_BB, 128), lambda i: (i, 0)),
        ],
        out_shape=[
            jax.ShapeDtypeStruct((_B, 128), jnp.int32),
            jax.ShapeDtypeStruct((_B, 128), jnp.int32),
        ],
    )(mask_f, adj2, next_op)
    return out[0][:, 0], out[1][:, 0]


def _sc_body(proc_hbm, adj_hbm, op_hbm, j_hbm, tr_hbm, act_hbm,
             opv, jv, pidx, tidx, procbuf, adjbuf, trbuf, actbuf, sem):
    wid = lax.axis_index("s") * _NC + lax.axis_index("c")
    base = wid * _BW
    pltpu.sync_copy(op_hbm.at[pl.ds(base, _BW)], opv)
    pltpu.sync_copy(j_hbm.at[pl.ds(base, _BW)], jv)

    lanes = lax.iota(jnp.int32, _L)
    for g in range(_NG):
        opl = opv[pl.ds(g * _L, _L)]
        fb = (base + g * _L + lanes) * (_NM * _NO) + opl
        tb = (base + g * _L + lanes) * _NT
        for m in range(_NM):
            p = m * _BW + g * _L
            pidx[p // 128, pl.ds(p % 128, _L)] = fb + m * _NO
        for t in range(_NT):
            p = t * _BW + g * _L
            tidx[p // 128, pl.ds(p % 128, _L)] = tb + t

    copies = []
    for r in range(_NM * _BW // 128):
        copies.append(pltpu.async_copy(
            proc_hbm.at[pidx.at[r]], procbuf.at[pl.ds(r * 128, 128)], sem))
        copies.append(pltpu.async_copy(
            adj_hbm.at[pidx.at[r]], adjbuf.at[pl.ds(r * 128, 128)], sem))
    for r in range(_NT * _BW // 128):
        copies.append(pltpu.async_copy(
            tr_hbm.at[tidx.at[r]], trbuf.at[pl.ds(r * 128, 128)], sem))
    for c in copies:
        c.wait()

    for g in range(_NG):
        bestp = jnp.full((_L,), jnp.inf, jnp.float32)
        bestm = jnp.zeros((_L,), jnp.int32)
        for m in range(_NM):
            p = m * _BW + g * _L
            pv = jnp.where(adjbuf[pl.ds(p, _L)] == 0, jnp.inf,
                           procbuf[pl.ds(p, _L)])
            lt = pv < bestp
            bestm = jnp.where(lt, jnp.full((_L,), m, jnp.int32), bestm)
            bestp = jnp.where(lt, pv, bestp)
        bestt = jnp.zeros((_L,), jnp.int32)
        besttv = jnp.full((_L,), jnp.inf, jnp.float32)
        for t in range(_NT):
            tv = trbuf[pl.ds(t * _BW + g * _L, _L)]
            lt = tv < besttv
            bestt = jnp.where(lt, jnp.full((_L,), t, jnp.int32), bestt)
            besttv = jnp.where(lt, tv, besttv)
        jvv = jv[pl.ds(g * _L, _L)]
        actbuf[pl.ds(g * _L, _L)] = (
            1 + jvv * (_NM * _NT) + bestm * _NT + bestt)

    pltpu.sync_copy(actbuf, act_hbm.at[pl.ds(base, _BW)])


def _sc_select(proc_flat, adj_flat, op1, j1, tr_flat):
    k = functools.partial(
        pl.kernel,
        out_type=jax.ShapeDtypeStruct((_B,), jnp.int32),
        mesh=plsc.VectorSubcoreMesh(core_axis_name="c", subcore_axis_name="s"),
        scratch_types=[
            pltpu.VMEM((_BW,), jnp.int32),
            pltpu.VMEM((_BW,), jnp.int32),
            pltpu.VMEM((_NM * _BW // 128, 128), jnp.int32),
            pltpu.VMEM((_NT * _BW // 128, 128), jnp.int32),
            pltpu.VMEM((_NM * _BW,), jnp.float32),
            pltpu.VMEM((_NM * _BW,), jnp.int32),
            pltpu.VMEM((_NT * _BW,), jnp.float32),
            pltpu.VMEM((_BW,), jnp.int32),
            pltpu.SemaphoreType.DMA,
        ],
    )(_sc_body)
    return k(proc_flat, adj_flat, op1, j1, tr_flat)


def _onehot_body(act_ref, out_ref):
    col = lax.broadcasted_iota(jnp.int32, (_BL, _NACT), 1)
    out_ref[...] = jnp.where(col == act_ref[...], 1.0, 0.0)


def _onehot(act2d):
    return pl.pallas_call(
        _onehot_body,
        grid=(_B // _BL,),
        in_specs=[pl.BlockSpec((_BL, 1), lambda i: (i, 0))],
        out_specs=pl.BlockSpec((_BL, _NACT), lambda i: (i, 0)),
        out_shape=jax.ShapeDtypeStruct((_B, _NACT), jnp.float32),
    )(act2d)


def kernel(job_done, machine_busy_until, truck_location, job_ops_adj,
           op_scheduled, next_op, proc_times, ops_ma_adj,
           truck_busy_until, action_mask):
    mask_f = jnp.logical_not(op_scheduled).astype(jnp.float32)
    adj2 = job_ops_adj.reshape(_B * _NJ, _NO)
    op1, j1 = _job_select(mask_f, adj2, next_op)
    act = _sc_select(
        proc_times.reshape(-1),
        ops_ma_adj.reshape(-1),
        op1, j1,
        truck_busy_until.reshape(-1),
    )
    logits = _onehot(act.reshape(_B, 1))
    return (logits, action_mask)


# speedup vs baseline: 1.5938x; 1.0358x over previous
import functools

import jax
import jax.numpy as jnp
from jax import lax
from jax.experimental import pallas as pl
from jax.experimental.pallas import tpu as pltpu
from jax.experimental.pallas import tpu_sc as plsc

_B = 2048
_NJ = 50
_NM = 20
_NT = 10
_NO = 500
_NACT = 1 + _NJ * _NM * _NT

_NC = 2
_NS = 16
_L = 16
_NW = _NC * _NS
_BWS = 128

_BB = 64
_BL = 64


def _job_body(m_ref, a_ref, nop_ref, op_ref, j_ref):
    mfull = m_ref[...]
    best = jnp.full((_BB, 1, 1), -jnp.inf, jnp.float32)
    sj = jnp.zeros((_BB, 1, 1), jnp.int32)
    ov = jnp.zeros((_BB, 1, 1), jnp.int32)
    for j0 in range(0, _NJ, 8):
        jw = min(8, _NJ - j0)
        r = jnp.zeros((_BB, jw, 1), jnp.float32)
        for o0 in range(0, _NO, 128):
            ow = min(128, _NO - o0)
            a = a_ref[:, j0:j0 + jw, o0:o0 + ow]
            r = r + jnp.sum(a * mfull[:, :, o0:o0 + ow], axis=2,
                            keepdims=True)
        jid = j0 + lax.broadcasted_iota(jnp.int32, (_BB, jw, 1), 1)
        m1 = jnp.max(r, axis=1, keepdims=True)
        j1 = jnp.min(jnp.where(r == m1, jid, _NJ), axis=1, keepdims=True)
        o1 = jnp.sum(jnp.where(jid == j1, nop_ref[:, j0:j0 + jw, :], 0),
                     axis=1, keepdims=True)
        upd = m1 > best
        best = jnp.where(upd, m1, best)
        sj = jnp.where(upd, j1, sj)
        ov = jnp.where(upd, o1, ov)
    op_ref[...] = jnp.broadcast_to(ov, (_BB, 1, 128))
    j_ref[...] = jnp.broadcast_to(sj, (_BB, 1, 128))


def _job_select(mask_f, job_ops_adj, next_op):
    out = pl.pallas_call(
        _job_body,
        grid=(_B // _BB,),
        in_specs=[
            pl.BlockSpec((_BB, 1, _NO), lambda i: (i, 0, 0)),
            pl.BlockSpec((_BB, _NJ, _NO), lambda i: (i, 0, 0)),
            pl.BlockSpec((_BB, _NJ, 1), lambda i: (i, 0, 0)),
        ],
        out_specs=[
            pl.BlockSpec((_BB, 1, 128), lambda i: (i, 0, 0)),
            pl.BlockSpec((_BB, 1, 128), lambda i: (i, 0, 0)),
        ],
        out_shape=[
            jax.ShapeDtypeStruct((_B, 1, 128), jnp.int32),
            jax.ShapeDtypeStruct((_B, 1, 128), jnp.int32),
        ],
    )(mask_f, job_ops_adj, next_op)
    return out


def _gather_body(op_ref, p_ref, v_ref, pv_ref):
    op3 = op_ref[...][:, :, :1]
    cols = []
    for m0 in range(0, _NM, 8):
        mw = min(8, _NM - m0)
        acc = jnp.zeros((_BB, mw), jnp.float32)
        for o0 in range(0, _NO, 128):
            ow = min(128, _NO - o0)
            oid = o0 + lax.broadcasted_iota(jnp.int32, (_BB, 1, ow), 2)
            oh = jnp.where(oid == op3, 1.0, 0.0)
            p = p_ref[:, m0:m0 + mw, o0:o0 + ow]
            v = v_ref[:, m0:m0 + mw, o0:o0 + ow]
            pm = jnp.where(v == 0, 1e30, p)
            acc = acc + jnp.sum(pm * oh, axis=2)
        cols.append(acc)
    pv_ref[...] = jnp.concatenate(cols, axis=1)


def _gather_cols(op_b, proc_times, ops_ma_adj):
    return pl.pallas_call(
        _gather_body,
        grid=(_B // _BB,),
        in_specs=[
            pl.BlockSpec((_BB, 1, 128), lambda i: (i, 0, 0)),
            pl.BlockSpec((_BB, _NM, _NO), lambda i: (i, 0, 0)),
            pl.BlockSpec((_BB, _NM, _NO), lambda i: (i, 0, 0)),
        ],
        out_specs=pl.BlockSpec((_BB, _NM), lambda i: (i, 0)),
        out_shape=jax.ShapeDtypeStruct((_B, _NM), jnp.float32),
    )(op_b, proc_times, ops_ma_adj)


def _sc_body(pvt_hbm, trt_hbm, j_hbm, act_hbm, pvb, trb, jb, actb):
    wid = lax.axis_index("s") * _NC + lax.axis_index("c")
    base = wid * _BWS

    @pl.when(wid < _B // _BWS)
    def _():
        pltpu.sync_copy(pvt_hbm.at[:, pl.ds(base, _BWS)], pvb)
        pltpu.sync_copy(trt_hbm.at[:, pl.ds(base, _BWS)], trb)
        pltpu.sync_copy(j_hbm.at[pl.ds(base, _BWS)], jb)

        for g in range(_BWS // _L):
            bestp = jnp.full((_L,), jnp.inf, jnp.float32)
            bestm = jnp.zeros((_L,), jnp.int32)
            for m in range(_NM):
                pv = pvb[m, pl.ds(g * _L, _L)]
                lt = pv < bestp
                bestm = jnp.where(lt, jnp.full((_L,), m, jnp.int32), bestm)
                bestp = jnp.where(lt, pv, bestp)
            bestt = jnp.zeros((_L,), jnp.int32)
            besttv = jnp.full((_L,), jnp.inf, jnp.float32)
            for t in range(_NT):
                tv = trb[t, pl.ds(g * _L, _L)]
                lt = tv < besttv
                bestt = jnp.where(lt, jnp.full((_L,), t, jnp.int32), bestt)
                besttv = jnp.where(lt, tv, besttv)
            jvv = jb[pl.ds(g * _L, _L)]
            actb[pl.ds(g * _L, _L)] = (
                1 + jvv * (_NM * _NT) + bestm * _NT + bestt)

        pltpu.sync_copy(actb, act_hbm.at[pl.ds(base, _BWS)])


def _sc_select(pvt, trt, j1):
    k = functools.partial(
        pl.kernel,
        out_type=jax.ShapeDtypeStruct((_B,), jnp.int32),
        mesh=plsc.VectorSubcoreMesh(core_axis_name="c", subcore_axis_name="s"),
        scratch_types=[
            pltpu.VMEM((_NM, _BWS), jnp.float32),
            pltpu.VMEM((_NT, _BWS), jnp.float32),
            pltpu.VMEM((_BWS,), jnp.int32),
            pltpu.VMEM((_BWS,), jnp.int32),
        ],
    )(_sc_body)
    return k(pvt, trt, j1)


def _onehot_body(act_ref, out_ref):
    col = lax.broadcasted_iota(jnp.int32, (_BL, _NACT), 1)
    out_ref[...] = jnp.where(col == act_ref[...], 1.0, 0.0)


def _onehot(act2d):
    return pl.pallas_call(
        _onehot_body,
        grid=(_B // _BL,),
        in_specs=[pl.BlockSpec((_BL, 1), lambda i: (i, 0))],
        out_specs=pl.BlockSpec((_BL, _NACT), lambda i: (i, 0)),
        out_shape=jax.ShapeDtypeStruct((_B, _NACT), jnp.float32),
    )(act2d)


def kernel(job_done, machine_busy_until, truck_location, job_ops_adj,
           op_scheduled, next_op, proc_times, ops_ma_adj,
           truck_busy_until, action_mask):
    mask_f = jnp.logical_not(op_scheduled).astype(jnp.float32)
    op_b, j_b = _job_select(mask_f.reshape(_B, 1, _NO),
                            job_ops_adj, next_op.reshape(_B, _NJ, 1))
    pv = _gather_cols(op_b, proc_times, ops_ma_adj)
    act = _sc_select(pv.T, truck_busy_until.T, j_b[:, 0, 0])
    logits = _onehot(act.reshape(_B, 1))
    return (logits, action_mask)


# speedup vs baseline: 1.6151x; 1.0134x over previous
import functools

import jax
import jax.numpy as jnp
from jax import lax
from jax.experimental import pallas as pl
from jax.experimental.pallas import tpu as pltpu
from jax.experimental.pallas import tpu_sc as plsc

_B = 2048
_NJ = 50
_NM = 20
_NT = 10
_NO = 500
_NACT = 1 + _NJ * _NM * _NT

_NC = 2
_NS = 16
_L = 16
_NW = _NC * _NS
_BW = _B // _NW
_RB = 4

_BB = 64
_BL = 64


def _job_body(m_ref, a_ref, nop_ref, t_ref, op_ref, j_ref, st_ref):
    mfull = m_ref[...]
    best = jnp.full((_BB, 1, 1), -jnp.inf, jnp.float32)
    sj = jnp.zeros((_BB, 1, 1), jnp.int32)
    ov = jnp.zeros((_BB, 1, 1), jnp.int32)
    for j0 in range(0, _NJ, 8):
        jw = min(8, _NJ - j0)
        r = jnp.zeros((_BB, jw, 1), jnp.float32)
        for o0 in range(0, _NO, 128):
            ow = min(128, _NO - o0)
            a = a_ref[:, j0:j0 + jw, o0:o0 + ow]
            r = r + jnp.sum(a * mfull[:, :, o0:o0 + ow], axis=2,
                            keepdims=True)
        jid = j0 + lax.broadcasted_iota(jnp.int32, (_BB, jw, 1), 1)
        m1 = jnp.max(r, axis=1, keepdims=True)
        j1 = jnp.min(jnp.where(r == m1, jid, _NJ), axis=1, keepdims=True)
        o1 = jnp.sum(jnp.where(jid == j1, nop_ref[:, j0:j0 + jw, :], 0),
                     axis=1, keepdims=True)
        upd = m1 > best
        best = jnp.where(upd, m1, best)
        sj = jnp.where(upd, j1, sj)
        ov = jnp.where(upd, o1, ov)
    op_ref[...] = jnp.broadcast_to(ov, (_BB, 1, 128))
    j_ref[...] = jnp.broadcast_to(sj, (_BB, 1, 128))
    tb = t_ref[...]
    tmin = jnp.min(tb, axis=1, keepdims=True)
    tid = lax.broadcasted_iota(jnp.int32, (_BB, _NT), 1)
    selt = jnp.min(jnp.where(tb == tmin, tid, _NT), axis=1, keepdims=True)
    st_ref[...] = jnp.broadcast_to(selt, (_BB, 128))


def _job_select(mask_f, job_ops_adj, next_op, truck_busy_until):
    out = pl.pallas_call(
        _job_body,
        grid=(_B // _BB,),
        in_specs=[
            pl.BlockSpec((_BB, 1, _NO), lambda i: (i, 0, 0)),
            pl.BlockSpec((_BB, _NJ, _NO), lambda i: (i, 0, 0)),
            pl.BlockSpec((_BB, _NJ, 1), lambda i: (i, 0, 0)),
            pl.BlockSpec((_BB, _NT), lambda i: (i, 0)),
        ],
        out_specs=[
            pl.BlockSpec((_BB, 1, 128), lambda i: (i, 0, 0)),
            pl.BlockSpec((_BB, 1, 128), lambda i: (i, 0, 0)),
            pl.BlockSpec((_BB, 128), lambda i: (i, 0)),
        ],
        out_shape=[
            jax.ShapeDtypeStruct((_B, 1, 128), jnp.int32),
            jax.ShapeDtypeStruct((_B, 1, 128), jnp.int32),
            jax.ShapeDtypeStruct((_B, 128), jnp.int32),
        ],
    )(mask_f, job_ops_adj, next_op, truck_busy_until)
    return out


def _sc_body(proc_hbm, adj_hbm, op_hbm, j_hbm, st_hbm, act_hbm,
             opv, jv, stv, pbuf, abuf, actb, sem):
    wid = lax.axis_index("s") * _NC + lax.axis_index("c")
    base = wid * _BW
    pltpu.sync_copy(op_hbm.at[pl.ds(base, _BW)], opv)
    pltpu.sync_copy(j_hbm.at[pl.ds(base, _BW)], jv)
    pltpu.sync_copy(st_hbm.at[pl.ds(base, _BW)], stv)

    def opscal(bl):
        v = opv[pl.ds((bl // _L) * _L, _L)]
        return v[bl % _L]

    def fire(r):
        hs = []
        slot = r % 2
        for b in range(_RB):
            bl = r * _RB + b
            ot = opscal(bl) // 128
            ws = pl.multiple_of(ot * 128, 128)
            bg = base + bl
            hs.append(pltpu.async_copy(
                proc_hbm.at[bg, :, pl.ds(ws, 128)],
                pbuf.at[slot, b, :, pl.ds(0, 128)], sem))
            hs.append(pltpu.async_copy(
                adj_hbm.at[bg, :, pl.ds(ws, 128)],
                abuf.at[slot, b, :, pl.ds(0, 128)], sem))
        return hs

    nr = _BW // _RB
    pend = {0: fire(0), 1: fire(1)}

    def process(r, acc):
        slot = r % 2
        lane = lax.iota(jnp.int32, _L)
        for b in range(_RB):
            bl = r * _RB + b
            c = opscal(bl) % 128
            c16 = pl.multiple_of((c // _L) * _L, _L)
            cmod = c % _L

            def mstep(m, carry):
                bpv, bmv = carry
                p16 = pbuf[slot, b, m, pl.ds(c16, _L)]
                v16 = abuf[slot, b, m, pl.ds(c16, _L)]
                pm16 = jnp.where(v16 == 0, jnp.float32(1e30), p16)
                lt = pm16 < bpv
                return (jnp.where(lt, pm16, bpv),
                        jnp.where(lt, jnp.full((_L,), m, jnp.int32), bmv))

            _, bmv = lax.fori_loop(
                0, _NM, mstep,
                (jnp.full((_L,), jnp.inf, jnp.float32),
                 jnp.zeros((_L,), jnp.int32)))
            bm_b = jnp.asarray(bmv).at[
                jnp.full((_L,), cmod, jnp.int32)].get(
                    mode="promise_in_bounds")

            j16 = jv[pl.ds((bl // _L) * _L, _L)]
            st16 = stv[pl.ds((bl // _L) * _L, _L)]
            act = (1 + j16[bl % _L] * (_NM * _NT) + bm_b * _NT
                   + st16[bl % _L])
            acc = jnp.where(lane == (bl % _L), act, acc)
            if bl % _L == _L - 1:
                actb[pl.ds((bl // _L) * _L, _L)] = acc
                acc = jnp.zeros((_L,), jnp.int32)
        return acc

    acc = jnp.zeros((_L,), jnp.int32)
    for r in range(nr):
        for h in pend[r % 2]:
            h.wait()
        acc = process(r, acc)
        if r + 2 < nr:
            pend[r % 2] = fire(r + 2)

    pltpu.sync_copy(actb, act_hbm.at[pl.ds(base, _BW)])


def _sc_select(proc_times, ops_ma_adj, op1, j1, st1):
    k = functools.partial(
        pl.kernel,
        out_type=jax.ShapeDtypeStruct((_B,), jnp.int32),
        mesh=plsc.VectorSubcoreMesh(core_axis_name="c", subcore_axis_name="s"),
        scratch_types=[
            pltpu.VMEM((_BW,), jnp.int32),
            pltpu.VMEM((_BW,), jnp.int32),
            pltpu.VMEM((_BW,), jnp.int32),
            pltpu.VMEM((2, _RB, _NM, 128), jnp.float32),
            pltpu.VMEM((2, _RB, _NM, 128), jnp.int32),
            pltpu.VMEM((_BW,), jnp.int32),
            pltpu.SemaphoreType.DMA,
        ],
    )(_sc_body)
    return k(proc_times, ops_ma_adj, op1, j1, st1)


def _onehot_body(act_ref, out_ref):
    col = lax.broadcasted_iota(jnp.int32, (_BL, _NACT), 1)
    out_ref[...] = jnp.where(col == act_ref[...], 1.0, 0.0)


def _onehot(act2d):
    return pl.pallas_call(
        _onehot_body,
        grid=(_B // _BL,),
        in_specs=[pl.BlockSpec((_BL, 1), lambda i: (i, 0))],
        out_specs=pl.BlockSpec((_BL, _NACT), lambda i: (i, 0)),
        out_shape=jax.ShapeDtypeStruct((_B, _NACT), jnp.float32),
    )(act2d)


def kernel(job_done, machine_busy_until, truck_location, job_ops_adj,
           op_scheduled, next_op, proc_times, ops_ma_adj,
           truck_busy_until, action_mask):
    mask_f = jnp.logical_not(op_scheduled).astype(jnp.float32)
    op_b, j_b, st_b = _job_select(mask_f.reshape(_B, 1, _NO),
                                  job_ops_adj, next_op.reshape(_B, _NJ, 1),
                                  truck_busy_until)
    act = _sc_select(proc_times, ops_ma_adj,
                     op_b[:, 0, 0], j_b[:, 0, 0], st_b[:, 0])
    logits = _onehot(act.reshape(_B, 1))
    return (logits, action_mask)
